# Initial kernel scaffold; baseline (speedup 1.0000x reference)
import jax
import jax.numpy as jnp
from jax.experimental import pallas as pl


def kernel(input_ids, token_type_ids, word_emb, pos_emb, type_emb, gamma, beta):
    raise NotImplementedError("write your pallas kernel here")


# speedup vs baseline: 2.0365x; 2.0365x over previous
import functools

import jax
import jax.numpy as jnp
from jax import lax
from jax.experimental import pallas as pl
from jax.experimental.pallas import tpu as pltpu
from jax.experimental.pallas import tpu_sc as plsc

D = 768
B = 4
S = 2048
TOKENS = B * S
EPS = 1e-5

NC, NS = 2, 16
NW = NC * NS
PER_W = TOKENS // NW
CH = 64
NCH = PER_W // CH

@functools.lru_cache(maxsize=1)
def _make_gather_rows():
    mesh = plsc.VectorSubcoreMesh(
        core_axis_name="c", subcore_axis_name="s", num_cores=NC, num_subcores=NS
    )

    @functools.partial(
        pl.kernel,
        mesh=mesh,
        out_type=jax.ShapeDtypeStruct((TOKENS, D), jnp.float32),
        scratch_types=[
            pltpu.VMEM((NCH, CH), jnp.int32),
            pltpu.VMEM((CH, D), jnp.float32),
            pltpu.VMEM((CH, D), jnp.float32),
            pltpu.SemaphoreType.DMA,
            pltpu.SemaphoreType.DMA,
        ],
    )
    def _gather_rows(ids_hbm, word_hbm, out_hbm, idx_v, buf0, buf1, sem0, sem1):
        wid = lax.axis_index("s") * NC + lax.axis_index("c")
        base = wid * PER_W
        pltpu.sync_copy(ids_hbm.at[wid], idx_v)
        bufs = (buf0, buf1)
        sems = (sem0, sem1)
        gathers = [None] * NCH
        gathers[0] = pltpu.async_copy(word_hbm.at[idx_v.at[0]], bufs[0], sems[0])
        for i in range(NCH):
            gathers[i].wait()
            if i + 1 < NCH:
                gathers[i + 1] = pltpu.async_copy(
                    word_hbm.at[idx_v.at[i + 1]], bufs[(i + 1) % 2], sems[(i + 1) % 2]
                )
            pltpu.sync_copy(bufs[i % 2], out_hbm.at[pl.ds(base + i * CH, CH)])

    return _gather_rows


ROWS_BLK = 256
GRID = TOKENS // ROWS_BLK
POS_BLKS = S // ROWS_BLK


def _ln_body(g_ref, pos_ref, tt_ref, type_ref, gamma_ref, beta_ref, o_ref):
    x = g_ref[...] + pos_ref[...]
    tt = tt_ref[0, 0, :].astype(jnp.float32)[:, None]
    t0 = type_ref[0:1, :]
    t1 = type_ref[1:2, :]
    x = x + t0 + tt * (t1 - t0)
    mu = jnp.mean(x, axis=1, keepdims=True)
    xc = x - mu
    var = jnp.mean(xc * xc, axis=1, keepdims=True)
    y = xc * lax.rsqrt(var + EPS)
    o_ref[...] = y * gamma_ref[...] + beta_ref[...]


_ln_call = pl.pallas_call(
    _ln_body,
    grid=(GRID,),
    in_specs=[
        pl.BlockSpec((ROWS_BLK, D), lambda t: (t, 0)),
        pl.BlockSpec((ROWS_BLK, D), lambda t: (t % POS_BLKS, 0)),
        pl.BlockSpec((1, 1, ROWS_BLK), lambda t: (t, 0, 0)),
        pl.BlockSpec((2, D), lambda t: (0, 0)),
        pl.BlockSpec((1, D), lambda t: (0, 0)),
        pl.BlockSpec((1, D), lambda t: (0, 0)),
    ],
    out_specs=pl.BlockSpec((ROWS_BLK, D), lambda t: (t, 0)),
    out_shape=jax.ShapeDtypeStruct((TOKENS, D), jnp.float32),
)


def kernel(input_ids, token_type_ids, word_emb, pos_emb, type_emb, gamma, beta):
    ids = input_ids.astype(jnp.int32).reshape(NW, NCH, CH)
    gathered = _make_gather_rows()(ids, word_emb)
    tt = token_type_ids.astype(jnp.int32).reshape(GRID, 1, ROWS_BLK)
    out = _ln_call(
        gathered,
        pos_emb,
        tt,
        type_emb,
        gamma.reshape(1, D),
        beta.reshape(1, D),
    )
    return out.reshape(B, S, D)


# speedup vs baseline: 2.3879x; 1.1726x over previous
import functools

import jax
import jax.numpy as jnp
from jax import lax
from jax.experimental import pallas as pl
from jax.experimental.pallas import tpu as pltpu
from jax.experimental.pallas import tpu_sc as plsc

D = 768
B = 4
S = 2048
TOKENS = B * S
EPS = 1e-5

NC, NS = 2, 16
NW = NC * NS
PER_W = TOKENS // NW
CH = 64
NCH = PER_W // CH

@functools.lru_cache(maxsize=1)
def _make_gather_rows():
    mesh = plsc.VectorSubcoreMesh(
        core_axis_name="c", subcore_axis_name="s", num_cores=NC, num_subcores=NS
    )

    @functools.partial(
        pl.kernel,
        mesh=mesh,
        out_type=jax.ShapeDtypeStruct((TOKENS, D), jnp.float32),
        scratch_types=[
            pltpu.VMEM((NCH, CH), jnp.int32),
            pltpu.VMEM((CH, D), jnp.float32),
            pltpu.VMEM((CH, D), jnp.float32),
            pltpu.SemaphoreType.DMA,
            pltpu.SemaphoreType.DMA,
        ],
    )
    def _gather_rows(ids_hbm, word_hbm, out_hbm, idx_v, buf0, buf1, sem0, sem1):
        wid = lax.axis_index("s") * NC + lax.axis_index("c")
        base = wid * PER_W
        pltpu.sync_copy(ids_hbm.at[wid], idx_v)
        bufs = (buf0, buf1)
        sems = (sem0, sem1)
        gathers = [None] * NCH
        gathers[0] = pltpu.async_copy(word_hbm.at[idx_v.at[0]], bufs[0], sems[0])
        for i in range(NCH):
            gathers[i].wait()
            if i + 1 < NCH:
                gathers[i + 1] = pltpu.async_copy(
                    word_hbm.at[idx_v.at[i + 1]], bufs[(i + 1) % 2], sems[(i + 1) % 2]
                )
            pltpu.sync_copy(bufs[i % 2], out_hbm.at[pl.ds(base + i * CH, CH)])

    return _gather_rows


ROWS_BLK = 512
GRID = TOKENS // ROWS_BLK
POS_BLKS = S // ROWS_BLK


def _ln_body(g_ref, pos_ref, tt_ref, type_ref, gamma_ref, beta_ref, o_ref):
    x = g_ref[...] + pos_ref[...]
    tt = tt_ref[0, 0, :].astype(jnp.float32)[:, None]
    t0 = type_ref[0:1, :]
    t1 = type_ref[1:2, :]
    x = x + t0 + tt * (t1 - t0)
    mu = jnp.mean(x, axis=1, keepdims=True)
    xc = x - mu
    var = jnp.mean(xc * xc, axis=1, keepdims=True)
    y = xc * lax.rsqrt(var + EPS)
    o_ref[...] = y * gamma_ref[...] + beta_ref[...]


_ln_call = pl.pallas_call(
    _ln_body,
    grid=(POS_BLKS, B),
    in_specs=[
        pl.BlockSpec((ROWS_BLK, D), lambda sb, b: (b * POS_BLKS + sb, 0)),
        pl.BlockSpec((ROWS_BLK, D), lambda sb, b: (sb, 0)),
        pl.BlockSpec((1, 1, ROWS_BLK), lambda sb, b: (b * POS_BLKS + sb, 0, 0)),
        pl.BlockSpec((2, D), lambda sb, b: (0, 0)),
        pl.BlockSpec((1, D), lambda sb, b: (0, 0)),
        pl.BlockSpec((1, D), lambda sb, b: (0, 0)),
    ],
    out_specs=pl.BlockSpec((ROWS_BLK, D), lambda sb, b: (b * POS_BLKS + sb, 0)),
    out_shape=jax.ShapeDtypeStruct((TOKENS, D), jnp.float32),
)


def kernel(input_ids, token_type_ids, word_emb, pos_emb, type_emb, gamma, beta):
    ids = input_ids.astype(jnp.int32).reshape(NW, NCH, CH)
    gathered = _make_gather_rows()(ids, word_emb)
    tt = token_type_ids.astype(jnp.int32).reshape(GRID, 1, ROWS_BLK)
    out = _ln_call(
        gathered,
        pos_emb,
        tt,
        type_emb,
        gamma.reshape(1, D),
        beta.reshape(1, D),
    )
    return out.reshape(B, S, D)
